# baseline (speedup 1.0000x reference)
import jax
import jax.numpy as jnp
from jax import lax
from jax.experimental import pallas as pl
from jax.experimental.pallas import tpu as pltpu
from jax.experimental.pallas import tpu_sc as plsc

_N = 10000
_E = 320000
_H = 128

_NC = 2
_NS = 16
_NW = _NC * _NS

_SLICES = (32000, 64000, 64000, 96000, 64000)
_CH = 40

_CH_SC = 40
_RPT = 624
_RTAIL = _N - _NS * _RPT

_NBUF = 5
_L1 = 2
_L2 = 3


def _make_gather_body(ept, cpt):
    def _gather_body(xs_hbm, xr_hbm, sidx_hbm, ridx_hbm, gsum_hbm,
                     sidx_v, ridx_v, rows, g1sems, g2sems, wsems):
        c = lax.axis_index("c")
        s = lax.axis_index("s")
        wid = s * _NC + c
        base0 = wid * ept
        pltpu.sync_copy(sidx_hbm.at[pl.ds(base0, ept)], sidx_v)
        pltpu.sync_copy(ridx_hbm.at[pl.ds(base0, ept)], ridx_v)

        def start_g1(i, b):
            pltpu.async_copy(xs_hbm.at[sidx_v.at[pl.ds(i * _CH, _CH)]], rows[b],
                             g1sems.at[b])

        def wait_g1(b):
            pltpu.make_async_copy(xs_hbm.at[sidx_v.at[pl.ds(0, _CH)]], rows[b],
                                  g1sems.at[b]).wait()

        def start_g2(i, b):
            pltpu.async_copy(xr_hbm.at[ridx_v.at[pl.ds(i * _CH, _CH)]], rows[b],
                             g2sems.at[b], add=True)

        def wait_g2(b):
            pltpu.make_async_copy(xr_hbm.at[ridx_v.at[pl.ds(0, _CH)]], rows[b],
                                  g2sems.at[b]).wait()

        def start_wb(i, b):
            pltpu.async_copy(rows[b], gsum_hbm.at[pl.ds(base0 + i * _CH, _CH)],
                             wsems.at[b])

        def wait_wb(b):
            pltpu.make_async_copy(rows[b], gsum_hbm.at[pl.ds(base0, _CH)],
                                  wsems.at[b]).wait()

        def body(k, carry):
            for b in range(_NBUF):
                i = k * _NBUF + b

                @pl.when(k > 0)
                def _slot_guard(b=b):
                    wait_wb(b)

                start_g1(i, b)

                d2 = (b - _L1) % _NBUF
                if b >= _L1:
                    wait_g1(d2)
                    start_g2(i - _L1, d2)
                else:
                    @pl.when(k > 0)
                    def _drain2(i=i, d2=d2):
                        wait_g1(d2)
                        start_g2(i - _L1, d2)

                d3 = (b - _L2) % _NBUF
                if b >= _L2:
                    wait_g2(d3)
                    start_wb(i - _L2, d3)
                else:
                    @pl.when(k > 0)
                    def _drain3(i=i, d3=d3):
                        wait_g2(d3)
                        start_wb(i - _L2, d3)
            return carry

        lax.fori_loop(0, cpt // _NBUF, body, 0)
        for j in range(cpt - _L1, cpt):
            wait_g1(j % _NBUF)
            start_g2(j, j % _NBUF)
        for j in range(cpt - _L2, cpt):
            wait_g2(j % _NBUF)
            start_wb(j, j % _NBUF)
        for b in range(_NBUF):
            wait_wb(b)

    return _gather_body


def _sc_gather(xs, xr, sidx, ridx):
    esl = sidx.shape[0]
    ept = esl // _NW
    mesh = plsc.VectorSubcoreMesh(core_axis_name="c", subcore_axis_name="s")
    f = pl.kernel(
        _make_gather_body(ept, ept // _CH),
        out_type=jax.ShapeDtypeStruct((esl, _H), jnp.float32),
        mesh=mesh,
        scratch_types=(
            pltpu.VMEM((ept,), jnp.int32),
            pltpu.VMEM((ept,), jnp.int32),
            [pltpu.VMEM((_CH, _H), jnp.float32) for _ in range(_NBUF)],
            pltpu.SemaphoreType.DMA((_NBUF,)),
            pltpu.SemaphoreType.DMA((_NBUF,)),
            pltpu.SemaphoreType.DMA((_NBUF,)),
        ),
    )
    return f(xs, xr, sidx, ridx)


def _make_scatter_body(etot):
    cpt = etot // _NC // _NS // _CH_SC

    def _scatter_body(ue_hbm, ridx_hbm, z_hbm, out_hbm,
                      ibufs, ebufs, acc, gsems, asems, acc_sem):
        c = lax.axis_index("c")
        s = lax.axis_index("s")
        row0 = s * _RPT
        tail0 = _NS * _RPT
        pltpu.async_copy(z_hbm.at[pl.ds(0, _RPT)], acc.at[pl.ds(row0, _RPT)],
                         acc_sem).wait()

        @pl.when(s == _NS - 1)
        def _zero_tail():
            pltpu.sync_copy(z_hbm.at[pl.ds(0, _RTAIL)],
                            acc.at[pl.ds(tail0, _RTAIL)])

        plsc.subcore_barrier()
        base0 = c * (etot // _NC) + s * (etot // _NC // _NS)

        if True:
            def start_load(i, b):
                base = base0 + i * _CH_SC
                pltpu.async_copy(ridx_hbm.at[pl.ds(base, _CH_SC)], ibufs[b],
                                 gsems.at[b])
                pltpu.async_copy(ue_hbm.at[pl.ds(base, _CH_SC)], ebufs[b],
                                 gsems.at[b])

            def wait_load(b):
                pltpu.make_async_copy(ridx_hbm.at[pl.ds(base0, _CH_SC)],
                                      ibufs[b], gsems.at[b]).wait()
                pltpu.make_async_copy(ue_hbm.at[pl.ds(base0, _CH_SC)],
                                      ebufs[b], gsems.at[b]).wait()

            def start_add(b):
                pltpu.async_copy(ebufs[b], acc.at[ibufs[b]], asems.at[b],
                                 add=True)

            def wait_add(b):
                pltpu.make_async_copy(ebufs[b], acc.at[ibufs[b]],
                                      asems.at[b]).wait()

            def body(k, carry):
                for b in range(_NBUF):
                    i = k * _NBUF + b

                    @pl.when(k > 0)
                    def _slot_guard(b=b):
                        wait_add(b)

                    start_load(i, b)

                    db = (b - _L2) % _NBUF
                    if b >= _L2:
                        wait_load(db)
                        start_add(db)
                    else:
                        @pl.when(k > 0)
                        def _drain(db=db):
                            wait_load(db)
                            start_add(db)
                return carry

            lax.fori_loop(0, cpt // _NBUF, body, 0)
            for t in range(_L2):
                db = (cpt - _L2 + t) % _NBUF
                wait_load(db)
                start_add(db)
            for b in range(_NBUF):
                wait_add(b)

        plsc.subcore_barrier()
        pltpu.sync_copy(acc.at[pl.ds(row0, _RPT)],
                        out_hbm.at[pl.ds(c * _N + row0, _RPT)])

        @pl.when(s == _NS - 1)
        def _copy_tail():
            pltpu.sync_copy(acc.at[pl.ds(tail0, _RTAIL)],
                            out_hbm.at[pl.ds(c * _N + tail0, _RTAIL)])

    return _scatter_body


def _sc_scatter(ue, ridx, zrows):
    mesh = plsc.VectorSubcoreMesh(core_axis_name="c", subcore_axis_name="s")
    f = pl.kernel(
        _make_scatter_body(ue.shape[0]),
        out_type=jax.ShapeDtypeStruct((_NC * _N, _H), jnp.float32),
        mesh=mesh,
        scratch_types=(
            [pltpu.VMEM((_CH_SC,), jnp.int32) for _ in range(_NBUF)],
            [pltpu.VMEM((_CH_SC, _H), jnp.float32) for _ in range(_NBUF)],
            pltpu.VMEM_SHARED((_N, _H), jnp.float32),
            pltpu.SemaphoreType.DMA((_NBUF,)),
            pltpu.SemaphoreType.DMA((_NBUF,)),
            pltpu.SemaphoreType.DMA,
        ),
    )
    return f(ue, ridx, zrows)


def _pre_body(x_ref, w0s_ref, w0r_ref, xs_ref, xr_ref):
    x = x_ref[...]
    xs_ref[...] = jnp.dot(x, w0s_ref[...], preferred_element_type=jnp.float32)
    xr_ref[...] = jnp.dot(x, w0r_ref[...], preferred_element_type=jnp.float32)


def _layer_norm_block(h, g, b):
    m = jnp.mean(h, axis=-1, keepdims=True)
    cen = h - m
    v = jnp.mean(cen * cen, axis=-1, keepdims=True)
    return cen / jnp.sqrt(v + 1e-5) * g + b


def _edge_body(ea_ref, gsum_ref, w0e_ref, b0_ref, w1_ref, b1_ref,
               w2_ref, b2_ref, g_ref, bln_ref, out_ref):
    x0 = (jnp.dot(ea_ref[...], w0e_ref[...], preferred_element_type=jnp.float32)
          + gsum_ref[...] + b0_ref[...])
    h0 = jnp.maximum(x0, 0.0)
    h1 = jnp.maximum(
        jnp.dot(h0, w1_ref[...], preferred_element_type=jnp.float32) + b1_ref[...], 0.0)
    h2 = jnp.dot(h1, w2_ref[...], preferred_element_type=jnp.float32) + b2_ref[...]
    out_ref[...] = _layer_norm_block(h2, g_ref[...], bln_ref[...])


def _node_body(x_ref, pa0_ref, pa1_ref,
               w0a_ref, w0b_ref, b0_ref, w1_ref, b1_ref,
               w2_ref, b2_ref, g_ref, bln_ref, out_ref):
    aggr = pa0_ref[...] + pa1_ref[...]
    x0 = (jnp.dot(x_ref[...], w0a_ref[...], preferred_element_type=jnp.float32)
          + jnp.dot(aggr, w0b_ref[...], preferred_element_type=jnp.float32)
          + b0_ref[...])
    h0 = jnp.maximum(x0, 0.0)
    h1 = jnp.maximum(
        jnp.dot(h0, w1_ref[...], preferred_element_type=jnp.float32) + b1_ref[...], 0.0)
    h2 = jnp.dot(h1, w2_ref[...], preferred_element_type=jnp.float32) + b2_ref[...]
    out_ref[...] = _layer_norm_block(h2, g_ref[...], bln_ref[...])


def _row_spec(rows, index_map):
    return pl.BlockSpec((rows, _H), index_map)


_W_SPEC = pl.BlockSpec((_H, _H), lambda i: (0, 0))
_B_SPEC = pl.BlockSpec((1, _H), lambda i: (0, 0))

_BN = 1000
_BE = 4000


def _tc_pre(x, w0s, w0r):
    return pl.pallas_call(
        _pre_body,
        grid=(_N // _BN,),
        in_specs=[_row_spec(_BN, lambda i: (i, 0)), _W_SPEC, _W_SPEC],
        out_specs=[_row_spec(_BN, lambda i: (i, 0))] * 2,
        out_shape=[jax.ShapeDtypeStruct((_N, _H), jnp.float32)] * 2,
    )(x, w0s, w0r)


def _edge_body_acc(ue_in_ref, ea_ref, gsum_ref, w0e_ref, b0_ref, w1_ref,
                   b1_ref, w2_ref, b2_ref, g_ref, bln_ref, out_ref):
    del ue_in_ref
    _edge_body(ea_ref, gsum_ref, w0e_ref, b0_ref, w1_ref, b1_ref,
               w2_ref, b2_ref, g_ref, bln_ref, out_ref)


def _tc_edge_mlp(ea, gsum, row0, ue_prev, w0e, b0, w1, b1, w2, b2, g, bln):
    nblk = gsum.shape[0] // _BE
    blk0 = row0 // _BE
    easpec = _row_spec(_BE, lambda i, blk0=blk0: (i + blk0, 0))
    espec = _row_spec(_BE, lambda i: (i, 0))
    out_spec = _row_spec(_BE, lambda i, blk0=blk0: (i + blk0, 0))
    wspecs = [_W_SPEC, _B_SPEC, _W_SPEC, _B_SPEC, _W_SPEC, _B_SPEC,
              _B_SPEC, _B_SPEC]
    args = (ea, gsum, w0e, b0, w1, b1, w2, b2, g, bln)
    if ue_prev is None:
        return pl.pallas_call(
            _edge_body,
            grid=(nblk,),
            in_specs=[easpec, espec] + wspecs,
            out_specs=out_spec,
            out_shape=jax.ShapeDtypeStruct((_E, _H), jnp.float32),
        )(*args)
    return pl.pallas_call(
        _edge_body_acc,
        grid=(nblk,),
        in_specs=[pl.BlockSpec(memory_space=pltpu.MemorySpace.HBM),
                  easpec, espec] + wspecs,
        out_specs=out_spec,
        out_shape=jax.ShapeDtypeStruct((_E, _H), jnp.float32),
        input_output_aliases={0: 0},
    )(ue_prev, *args)


def _tc_node_mlp(x, pa, w0a, w0b, b0, w1, b1, w2, b2, g, bln):
    nspec = _row_spec(_BN, lambda i: (i, 0))
    p1spec = _row_spec(_BN, lambda i: (i + _N // _BN, 0))
    return pl.pallas_call(
        _node_body,
        grid=(_N // _BN,),
        in_specs=[nspec, nspec, p1spec,
                  _W_SPEC, _W_SPEC, _B_SPEC, _W_SPEC, _B_SPEC, _W_SPEC,
                  _B_SPEC, _B_SPEC, _B_SPEC],
        out_specs=nspec,
        out_shape=jax.ShapeDtypeStruct((_N, _H), jnp.float32),
    )(x, pa, pa, w0a, w0b, b0, w1, b1, w2, b2, g, bln)


def kernel(input_features, edge_index, edge_attr, ew0, eb0, ew1, eb1, ew2, eb2,
           eg, eb_ln, nw0, nb0, nw1, nb1, nw2, nb2, ng, nb_ln):
    senders = edge_index[0]
    receivers = edge_index[1]

    w0e = ew0[:_H]
    w0s = ew0[_H:2 * _H]
    w0r = ew0[2 * _H:]

    eb0r = eb0.reshape(1, _H)
    eb1r = eb1.reshape(1, _H)
    eb2r = eb2.reshape(1, _H)
    egr = eg.reshape(1, _H)
    eblnr = eb_ln.reshape(1, _H)

    xs, xr = _tc_pre(input_features, w0s, w0r)

    ue = None
    row0 = 0
    for esl in _SLICES:
        sidx = lax.slice(senders, (row0,), (row0 + esl,))
        ridx = lax.slice(receivers, (row0,), (row0 + esl,))
        gsum = _sc_gather(xs, xr, sidx, ridx)
        ue = _tc_edge_mlp(edge_attr, gsum, row0, ue, w0e,
                          eb0r, ew1, eb1r, ew2, eb2r, egr, eblnr)
        row0 += esl

    zrows = jnp.zeros((_RPT + _RTAIL, _H), jnp.float32)
    pa = _sc_scatter(ue, receivers, zrows)

    un = _tc_node_mlp(
        input_features, pa, nw0[:_H], nw0[_H:],
        nb0.reshape(1, _H), nw1, nb1.reshape(1, _H), nw2, nb2.reshape(1, _H),
        ng.reshape(1, _H), nb_ln.reshape(1, _H))

    return (un, ue)


# speedup vs baseline: 1.0165x; 1.0165x over previous
import jax
import jax.numpy as jnp
from jax import lax
from jax.experimental import pallas as pl
from jax.experimental.pallas import tpu as pltpu
from jax.experimental.pallas import tpu_sc as plsc

_N = 10000
_E = 320000
_H = 128

_NC = 2
_NS = 16
_NW = _NC * _NS

_SL = 5
_ESL = _E // _SL
_EPT = _ESL // _NW
_CH = 80
_CPT = _EPT // _CH

_CH_SC = 40
_CPT_SC = _ESL // _NC // _NS // _CH_SC
_RPT = 624
_RTAIL = _N - _NS * _RPT

_NBUF = 5
_L1 = 2
_L2 = 3


def _gather_body(xs_hbm, xr_hbm, sidx_hbm, ridx_hbm, gsum_hbm,
                 sidx_v, ridx_v, rows, g1sems, g2sems, wsems):
    c = lax.axis_index("c")
    s = lax.axis_index("s")
    wid = s * _NC + c
    base0 = wid * _EPT
    pltpu.sync_copy(sidx_hbm.at[pl.ds(base0, _EPT)], sidx_v)
    pltpu.sync_copy(ridx_hbm.at[pl.ds(base0, _EPT)], ridx_v)

    def start_g1(i, b):
        pltpu.async_copy(xs_hbm.at[sidx_v.at[pl.ds(i * _CH, _CH)]], rows[b],
                         g1sems.at[b])

    def wait_g1(b):
        pltpu.make_async_copy(xs_hbm.at[sidx_v.at[pl.ds(0, _CH)]], rows[b],
                              g1sems.at[b]).wait()

    def start_g2(i, b):
        pltpu.async_copy(xr_hbm.at[ridx_v.at[pl.ds(i * _CH, _CH)]], rows[b],
                         g2sems.at[b], add=True)

    def wait_g2(b):
        pltpu.make_async_copy(xr_hbm.at[ridx_v.at[pl.ds(0, _CH)]], rows[b],
                              g2sems.at[b]).wait()

    def start_wb(i, b):
        pltpu.async_copy(rows[b], gsum_hbm.at[pl.ds(base0 + i * _CH, _CH)],
                         wsems.at[b])

    def wait_wb(b):
        pltpu.make_async_copy(rows[b], gsum_hbm.at[pl.ds(base0, _CH)],
                              wsems.at[b]).wait()

    def body(k, carry):
        for b in range(_NBUF):
            i = k * _NBUF + b

            @pl.when(k > 0)
            def _slot_guard(b=b):
                wait_wb(b)

            start_g1(i, b)

            d2 = (b - _L1) % _NBUF
            if b >= _L1:
                wait_g1(d2)
                start_g2(i - _L1, d2)
            else:
                @pl.when(k > 0)
                def _drain2(i=i, d2=d2):
                    wait_g1(d2)
                    start_g2(i - _L1, d2)

            d3 = (b - _L2) % _NBUF
            if b >= _L2:
                wait_g2(d3)
                start_wb(i - _L2, d3)
            else:
                @pl.when(k > 0)
                def _drain3(i=i, d3=d3):
                    wait_g2(d3)
                    start_wb(i - _L2, d3)
        return carry

    lax.fori_loop(0, _CPT // _NBUF, body, 0)
    for j in range(_CPT - _L1, _CPT):
        wait_g1(j % _NBUF)
        start_g2(j, j % _NBUF)
    for j in range(_CPT - _L2, _CPT):
        wait_g2(j % _NBUF)
        start_wb(j, j % _NBUF)
    for b in range(_NBUF):
        wait_wb(b)


def _sc_gather(xs, xr, sidx, ridx):
    mesh = plsc.VectorSubcoreMesh(core_axis_name="c", subcore_axis_name="s")
    f = pl.kernel(
        _gather_body,
        out_type=jax.ShapeDtypeStruct((_ESL, _H), jnp.float32),
        mesh=mesh,
        scratch_types=(
            pltpu.VMEM((_EPT,), jnp.int32),
            pltpu.VMEM((_EPT,), jnp.int32),
            [pltpu.VMEM((_CH, _H), jnp.float32) for _ in range(_NBUF)],
            pltpu.SemaphoreType.DMA((_NBUF,)),
            pltpu.SemaphoreType.DMA((_NBUF,)),
            pltpu.SemaphoreType.DMA((_NBUF,)),
        ),
    )
    return f(xs, xr, sidx, ridx)


def _make_scatter_body(etot):
    cpt = etot // _NC // _NS // _CH_SC

    def _scatter_body(ue_hbm, ridx_hbm, z_hbm, out_hbm,
                      ibufs, ebufs, acc, gsems, asems, acc_sem):
        c = lax.axis_index("c")
        s = lax.axis_index("s")
        row0 = s * _RPT
        tail0 = _NS * _RPT
        pltpu.async_copy(z_hbm.at[pl.ds(0, _RPT)], acc.at[pl.ds(row0, _RPT)],
                         acc_sem).wait()

        @pl.when(s == _NS - 1)
        def _zero_tail():
            pltpu.sync_copy(z_hbm.at[pl.ds(0, _RTAIL)],
                            acc.at[pl.ds(tail0, _RTAIL)])

        plsc.subcore_barrier()
        base0 = c * (etot // _NC) + s * (etot // _NC // _NS)

        if True:
            def start_load(i, b):
                base = base0 + i * _CH_SC
                pltpu.async_copy(ridx_hbm.at[pl.ds(base, _CH_SC)], ibufs[b],
                                 gsems.at[b])
                pltpu.async_copy(ue_hbm.at[pl.ds(base, _CH_SC)], ebufs[b],
                                 gsems.at[b])

            def wait_load(b):
                pltpu.make_async_copy(ridx_hbm.at[pl.ds(base0, _CH_SC)],
                                      ibufs[b], gsems.at[b]).wait()
                pltpu.make_async_copy(ue_hbm.at[pl.ds(base0, _CH_SC)],
                                      ebufs[b], gsems.at[b]).wait()

            def start_add(b):
                pltpu.async_copy(ebufs[b], acc.at[ibufs[b]], asems.at[b],
                                 add=True)

            def wait_add(b):
                pltpu.make_async_copy(ebufs[b], acc.at[ibufs[b]],
                                      asems.at[b]).wait()

            def body(k, carry):
                for b in range(_NBUF):
                    i = k * _NBUF + b

                    @pl.when(k > 0)
                    def _slot_guard(b=b):
                        wait_add(b)

                    start_load(i, b)

                    db = (b - _L2) % _NBUF
                    if b >= _L2:
                        wait_load(db)
                        start_add(db)
                    else:
                        @pl.when(k > 0)
                        def _drain(db=db):
                            wait_load(db)
                            start_add(db)
                return carry

            lax.fori_loop(0, cpt // _NBUF, body, 0)
            for t in range(_L2):
                db = (cpt - _L2 + t) % _NBUF
                wait_load(db)
                start_add(db)
            for b in range(_NBUF):
                wait_add(b)

        plsc.subcore_barrier()
        pltpu.sync_copy(acc.at[pl.ds(row0, _RPT)],
                        out_hbm.at[pl.ds(c * _N + row0, _RPT)])

        @pl.when(s == _NS - 1)
        def _copy_tail():
            pltpu.sync_copy(acc.at[pl.ds(tail0, _RTAIL)],
                            out_hbm.at[pl.ds(c * _N + tail0, _RTAIL)])

    return _scatter_body


def _sc_scatter(ue, ridx, zrows):
    mesh = plsc.VectorSubcoreMesh(core_axis_name="c", subcore_axis_name="s")
    f = pl.kernel(
        _make_scatter_body(ue.shape[0]),
        out_type=jax.ShapeDtypeStruct((_NC * _N, _H), jnp.float32),
        mesh=mesh,
        scratch_types=(
            [pltpu.VMEM((_CH_SC,), jnp.int32) for _ in range(_NBUF)],
            [pltpu.VMEM((_CH_SC, _H), jnp.float32) for _ in range(_NBUF)],
            pltpu.VMEM_SHARED((_N, _H), jnp.float32),
            pltpu.SemaphoreType.DMA((_NBUF,)),
            pltpu.SemaphoreType.DMA((_NBUF,)),
            pltpu.SemaphoreType.DMA,
        ),
    )
    return f(ue, ridx, zrows)


def _pre_body(x_ref, w0s_ref, w0r_ref, xs_ref, xr_ref):
    x = x_ref[...]
    xs_ref[...] = jnp.dot(x, w0s_ref[...], preferred_element_type=jnp.float32)
    xr_ref[...] = jnp.dot(x, w0r_ref[...], preferred_element_type=jnp.float32)


def _layer_norm_block(h, g, b):
    m = jnp.mean(h, axis=-1, keepdims=True)
    cen = h - m
    v = jnp.mean(cen * cen, axis=-1, keepdims=True)
    return cen / jnp.sqrt(v + 1e-5) * g + b


def _edge_body(ea_ref, gsum_ref, w0e_ref, b0_ref, w1_ref, b1_ref,
               w2_ref, b2_ref, g_ref, bln_ref, out_ref):
    x0 = (jnp.dot(ea_ref[...], w0e_ref[...], preferred_element_type=jnp.float32)
          + gsum_ref[...] + b0_ref[...])
    h0 = jnp.maximum(x0, 0.0)
    h1 = jnp.maximum(
        jnp.dot(h0, w1_ref[...], preferred_element_type=jnp.float32) + b1_ref[...], 0.0)
    h2 = jnp.dot(h1, w2_ref[...], preferred_element_type=jnp.float32) + b2_ref[...]
    out_ref[...] = _layer_norm_block(h2, g_ref[...], bln_ref[...])


def _node_body(x_ref, pa0_ref, pa1_ref,
               w0a_ref, w0b_ref, b0_ref, w1_ref, b1_ref,
               w2_ref, b2_ref, g_ref, bln_ref, out_ref):
    aggr = pa0_ref[...] + pa1_ref[...]
    x0 = (jnp.dot(x_ref[...], w0a_ref[...], preferred_element_type=jnp.float32)
          + jnp.dot(aggr, w0b_ref[...], preferred_element_type=jnp.float32)
          + b0_ref[...])
    h0 = jnp.maximum(x0, 0.0)
    h1 = jnp.maximum(
        jnp.dot(h0, w1_ref[...], preferred_element_type=jnp.float32) + b1_ref[...], 0.0)
    h2 = jnp.dot(h1, w2_ref[...], preferred_element_type=jnp.float32) + b2_ref[...]
    out_ref[...] = _layer_norm_block(h2, g_ref[...], bln_ref[...])


def _row_spec(rows, index_map):
    return pl.BlockSpec((rows, _H), index_map)


_W_SPEC = pl.BlockSpec((_H, _H), lambda i: (0, 0))
_B_SPEC = pl.BlockSpec((1, _H), lambda i: (0, 0))

_BN = 1000
_BE = 4000


def _tc_pre(x, w0s, w0r):
    return pl.pallas_call(
        _pre_body,
        grid=(_N // _BN,),
        in_specs=[_row_spec(_BN, lambda i: (i, 0)), _W_SPEC, _W_SPEC],
        out_specs=[_row_spec(_BN, lambda i: (i, 0))] * 2,
        out_shape=[jax.ShapeDtypeStruct((_N, _H), jnp.float32)] * 2,
    )(x, w0s, w0r)


def _edge_body_acc(ue_in_ref, ea_ref, gsum_ref, w0e_ref, b0_ref, w1_ref,
                   b1_ref, w2_ref, b2_ref, g_ref, bln_ref, out_ref):
    del ue_in_ref
    _edge_body(ea_ref, gsum_ref, w0e_ref, b0_ref, w1_ref, b1_ref,
               w2_ref, b2_ref, g_ref, bln_ref, out_ref)


def _tc_edge_mlp(ea, gsum, sl, ue_prev, w0e, b0, w1, b1, w2, b2, g, bln):
    nblk = _ESL // _BE
    easpec = _row_spec(_BE, lambda i, sl=sl: (i + sl * nblk, 0))
    espec = _row_spec(_BE, lambda i: (i, 0))
    out_spec = _row_spec(_BE, lambda i, sl=sl: (i + sl * nblk, 0))
    wspecs = [_W_SPEC, _B_SPEC, _W_SPEC, _B_SPEC, _W_SPEC, _B_SPEC,
              _B_SPEC, _B_SPEC]
    args = (ea, gsum, w0e, b0, w1, b1, w2, b2, g, bln)
    if sl == 0:
        return pl.pallas_call(
            _edge_body,
            grid=(nblk,),
            in_specs=[easpec, espec] + wspecs,
            out_specs=out_spec,
            out_shape=jax.ShapeDtypeStruct((_E, _H), jnp.float32),
        )(*args)
    return pl.pallas_call(
        _edge_body_acc,
        grid=(nblk,),
        in_specs=[pl.BlockSpec(memory_space=pltpu.MemorySpace.HBM),
                  easpec, espec] + wspecs,
        out_specs=out_spec,
        out_shape=jax.ShapeDtypeStruct((_E, _H), jnp.float32),
        input_output_aliases={0: 0},
    )(ue_prev, *args)


def _tc_node_mlp(x, pa, w0a, w0b, b0, w1, b1, w2, b2, g, bln):
    nspec = _row_spec(_BN, lambda i: (i, 0))
    p1spec = _row_spec(_BN, lambda i: (i + _N // _BN, 0))
    return pl.pallas_call(
        _node_body,
        grid=(_N // _BN,),
        in_specs=[nspec, nspec, p1spec,
                  _W_SPEC, _W_SPEC, _B_SPEC, _W_SPEC, _B_SPEC, _W_SPEC,
                  _B_SPEC, _B_SPEC, _B_SPEC],
        out_specs=nspec,
        out_shape=jax.ShapeDtypeStruct((_N, _H), jnp.float32),
    )(x, pa, pa, w0a, w0b, b0, w1, b1, w2, b2, g, bln)


def kernel(input_features, edge_index, edge_attr, ew0, eb0, ew1, eb1, ew2, eb2,
           eg, eb_ln, nw0, nb0, nw1, nb1, nw2, nb2, ng, nb_ln):
    senders = edge_index[0]
    receivers = edge_index[1]

    w0e = ew0[:_H]
    w0s = ew0[_H:2 * _H]
    w0r = ew0[2 * _H:]

    eb0r = eb0.reshape(1, _H)
    eb1r = eb1.reshape(1, _H)
    eb2r = eb2.reshape(1, _H)
    egr = eg.reshape(1, _H)
    eblnr = eb_ln.reshape(1, _H)

    xs, xr = _tc_pre(input_features, w0s, w0r)

    ue = None
    for sl in range(_SL):
        sidx = lax.slice(senders, (sl * _ESL,), ((sl + 1) * _ESL,))
        ridx = lax.slice(receivers, (sl * _ESL,), ((sl + 1) * _ESL,))
        gsum = _sc_gather(xs, xr, sidx, ridx)
        ue = _tc_edge_mlp(edge_attr, gsum, sl, ue, w0e,
                          eb0r, ew1, eb1r, ew2, eb2r, egr, eblnr)

    zrows = jnp.zeros((_RPT + _RTAIL, _H), jnp.float32)
    pa = _sc_scatter(ue, receivers, zrows)

    un = _tc_node_mlp(
        input_features, pa, nw0[:_H], nw0[_H:],
        nb0.reshape(1, _H), nw1, nb1.reshape(1, _H), nw2, nb2.reshape(1, _H),
        ng.reshape(1, _H), nb_ln.reshape(1, _H))

    return (un, ue)
